# baseline (speedup 1.0000x reference)
import functools

import jax
import jax.numpy as jnp
from jax import lax
from jax.experimental import pallas as pl
from jax.experimental.pallas import tpu as pltpu
from jax.experimental.pallas import tpu_sc as plsc

_BATCH = 4096
_HIST = 50
_DIM = 128
_B = _BATCH * _HIST
_NC = 2
_NS = 16
_NW = _NC * _NS
_B_W = _B // _NW
_CHUNK = 200
_NCHUNK = _B_W // _CHUNK
_NBUF = 4

_mesh = plsc.VectorSubcoreMesh(core_axis_name="c", subcore_axis_name="s")


@functools.partial(
    pl.kernel,
    mesh=_mesh,
    out_type=jax.ShapeDtypeStruct((_B, _DIM), jnp.float32),
    scratch_types=[
        pltpu.VMEM((_B_W,), jnp.int32),
    ]
    + [pltpu.VMEM((_CHUNK, _DIM), jnp.float32) for _ in range(_NBUF)]
    + [pltpu.SemaphoreType.DMA for _ in range(2 * _NBUF)],
)
def _gather(x_hbm, table_hbm, out_hbm, idx_v, *bufs_and_sems):
    rows = bufs_and_sems[:_NBUF]
    gsem = bufs_and_sems[_NBUF:2 * _NBUF]
    ssem = bufs_and_sems[2 * _NBUF:]
    wid = lax.axis_index("s") * _NC + lax.axis_index("c")
    base = wid * _B_W
    pltpu.sync_copy(x_hbm.at[pl.ds(base, _B_W)], idx_v)

    def start_gather(j, b):
        idx = idx_v.at[pl.ds(j * _CHUNK, _CHUNK)]
        pltpu.make_async_copy(table_hbm.at[idx], rows[b], gsem[b]).start()

    def wait_gather(b):
        idx = idx_v.at[pl.ds(0, _CHUNK)]
        pltpu.make_async_copy(table_hbm.at[idx], rows[b], gsem[b]).wait()

    def start_store(j, b):
        dst = out_hbm.at[pl.ds(base + j * _CHUNK, _CHUNK)]
        pltpu.make_async_copy(rows[b], dst, ssem[b]).start()

    def wait_store(b):
        dst = out_hbm.at[pl.ds(base, _CHUNK)]
        pltpu.make_async_copy(rows[b], dst, ssem[b]).wait()

    for b in range(_NBUF):
        start_gather(b, b)

    def outer(t, carry):
        for b in range(_NBUF):
            wait_gather(b)
            start_store(t * _NBUF + b, b)

        @pl.when(t < _NCHUNK // _NBUF - 1)
        def _():
            for b in range(_NBUF):
                wait_store(b)
                start_gather((t + 1) * _NBUF + b, b)

        return carry

    lax.fori_loop(0, _NCHUNK // _NBUF, outer, 0)

    for b in range(_NBUF):
        wait_store(b)


def kernel(x, table):
    x1 = x.reshape(_B).astype(jnp.int32)
    out = _gather(x1, table)
    return out.reshape(_BATCH, _HIST, _DIM)


# speedup vs baseline: 1.7992x; 1.7992x over previous
import functools

import jax
import jax.numpy as jnp
from jax import lax
from jax.experimental import pallas as pl
from jax.experimental.pallas import tpu as pltpu
from jax.experimental.pallas import tpu_sc as plsc

_BATCH = 4096
_HIST = 50
_DIM = 128
_B = _BATCH * _HIST
_NC = 2
_NS = 16
_NW = _NC * _NS
_B_W = _B // _NW
_BB = 4
_CHUNK = _BB * _HIST
_NCHUNK = _B_W // _CHUNK
_NBUF = 4
_B_PW = _BATCH // _NW

_mesh = plsc.VectorSubcoreMesh(core_axis_name="c", subcore_axis_name="s")


@functools.partial(
    pl.kernel,
    mesh=_mesh,
    out_type=jax.ShapeDtypeStruct((_BATCH, _HIST, _DIM), jnp.float32),
    scratch_types=[
        pltpu.VMEM((_B_PW, _HIST), jnp.int32),
    ]
    + [pltpu.VMEM((_BB, _HIST, _DIM), jnp.float32) for _ in range(_NBUF)]
    + [pltpu.SemaphoreType.DMA for _ in range(2 * _NBUF)],
)
def _gather(x_hbm, table_hbm, out_hbm, idx_v, *bufs_and_sems):
    rows = bufs_and_sems[:_NBUF]
    gsem = bufs_and_sems[_NBUF:2 * _NBUF]
    ssem = bufs_and_sems[2 * _NBUF:]
    wid = lax.axis_index("s") * _NC + lax.axis_index("c")
    obase = wid * _B_PW
    pltpu.sync_copy(x_hbm.at[pl.ds(obase, _B_PW)], idx_v)

    def start_gather(j, b):
        for k in range(_BB):
            idx = idx_v.at[j * _BB + k]
            pltpu.make_async_copy(table_hbm.at[idx], rows[b].at[k], gsem[b]).start()

    def wait_gather(b):
        idx = idx_v.at[0]
        for k in range(_BB):
            pltpu.make_async_copy(table_hbm.at[idx], rows[b].at[k], gsem[b]).wait()

    def start_store(j, b):
        dst = out_hbm.at[pl.ds(obase + j * _BB, _BB)]
        pltpu.make_async_copy(rows[b], dst, ssem[b]).start()

    def wait_store(b):
        dst = out_hbm.at[pl.ds(obase, _BB)]
        pltpu.make_async_copy(rows[b], dst, ssem[b]).wait()

    for b in range(_NBUF):
        start_gather(b, b)

    def outer(t, carry):
        for b in range(_NBUF):
            wait_gather(b)
            start_store(t * _NBUF + b, b)

        @pl.when(t < _NCHUNK // _NBUF - 1)
        def _():
            for b in range(_NBUF):
                wait_store(b)
                start_gather((t + 1) * _NBUF + b, b)

        return carry

    lax.fori_loop(0, _NCHUNK // _NBUF, outer, 0)

    for b in range(_NBUF):
        wait_store(b)


def kernel(x, table):
    return _gather(x.astype(jnp.int32), table)
